# baseline (speedup 1.0000x reference)
import functools

import jax
import jax.numpy as jnp
from jax import lax
from jax.experimental import pallas as pl
from jax.experimental.pallas import tpu as pltpu
from jax.experimental.pallas import tpu_sc as plsc

NUM_CORES = 2
NUM_SUBCORES = 16
NW = NUM_CORES * NUM_SUBCORES
IDX_LANES = 128
GROUP = 4
ROWS_PER_GROUP = IDX_LANES * GROUP


def _sc_gather(table, idx2d, n_rows, d):
    per_w = n_rows // NW
    idx_rows = per_w // IDX_LANES
    groups = per_w // ROWS_PER_GROUP
    mesh = plsc.VectorSubcoreMesh(core_axis_name="c", subcore_axis_name="s")

    @functools.partial(
        pl.kernel,
        mesh=mesh,
        out_type=jax.ShapeDtypeStruct((n_rows, d), table.dtype),
        scratch_types=[
            pltpu.VMEM((idx_rows, IDX_LANES), jnp.int32),
            pltpu.VMEM((ROWS_PER_GROUP, d), table.dtype),
            pltpu.SemaphoreType.DMA,
        ],
    )
    def gather_kernel(table_hbm, idx_hbm, out_hbm, idx_v, rows_v, sem):
        wid = lax.axis_index("s") * NUM_CORES + lax.axis_index("c")
        row0 = wid * per_w
        pltpu.sync_copy(idx_hbm.at[pl.ds(wid * idx_rows, idx_rows)], idx_v)

        def body(g, carry):
            copies = [
                pltpu.make_async_copy(
                    table_hbm.at[idx_v.at[g * GROUP + b]],
                    rows_v.at[pl.ds(b * IDX_LANES, IDX_LANES)],
                    sem,
                )
                for b in range(GROUP)
            ]
            for c in copies:
                c.start()
            for c in copies:
                c.wait()
            pltpu.sync_copy(
                rows_v, out_hbm.at[pl.ds(row0 + g * ROWS_PER_GROUP, ROWS_PER_GROUP)])
            return carry

        lax.fori_loop(0, groups, body, 0)

    return gather_kernel(table, idx2d)


def _encoder(g3, enc_w, enc_b2d, bt=512, tc=40):
    b, ctx, e = g3.shape
    nb, nk = b // bt, ctx // tc

    def body(g_ref, w_ref, b_ref, out_ref, acc_ref):
        kk = pl.program_id(0)
        ii = pl.program_id(1)
        part = lax.dot_general(
            g_ref[:, 0, :].astype(jnp.bfloat16), w_ref[:, 0:e],
            (((1,), (1,)), ((), ())),
            preferred_element_type=jnp.float32)
        for j in range(1, tc):
            part += lax.dot_general(
                g_ref[:, j, :].astype(jnp.bfloat16), w_ref[:, j * e:(j + 1) * e],
                (((1,), (1,)), ((), ())),
                preferred_element_type=jnp.float32)
        sl = pl.ds(ii * bt, bt)

        @pl.when(kk == 0)
        def _():
            acc_ref[sl, :] = part

        @pl.when(kk > 0)
        def _():
            acc_ref[sl, :] = acc_ref[sl, :] + part

        @pl.when(kk == nk - 1)
        def _():
            out_ref[...] = acc_ref[sl, :] + b_ref[...]

    return pl.pallas_call(
        body,
        grid=(nk, nb),
        in_specs=[
            pl.BlockSpec((bt, tc, e), lambda kk, ii: (ii, kk, 0)),
            pl.BlockSpec((e, tc * e), lambda kk, ii: (0, kk)),
            pl.BlockSpec((1, e), lambda kk, ii: (0, 0)),
        ],
        out_specs=pl.BlockSpec((bt, e), lambda kk, ii: (ii, 0)),
        out_shape=jax.ShapeDtypeStruct((b, e), jnp.float32),
        scratch_shapes=[pltpu.VMEM((b, e), jnp.float32)],
    )(g3, enc_w, enc_b2d)


def _decoder(encoded, dec_w, dec_b2d, bt=512, tc=40):
    b, e = encoded.shape
    k = dec_w.shape[0]
    ctx = k // e
    nb, nn = b // bt, ctx // tc
    nc = tc * e

    def body(enc_ref, w_ref, b_ref, out_ref):
        enc = enc_ref[...]
        for j in range(tc):
            res = lax.dot_general(
                enc, w_ref[j * e:(j + 1) * e, :], (((1,), (1,)), ((), ())),
                preferred_element_type=jnp.float32)
            out_ref[:, j, :] = res + b_ref[0:1, j * e:(j + 1) * e]

    return pl.pallas_call(
        body,
        grid=(nn, nb),
        in_specs=[
            pl.BlockSpec((bt, e), lambda nn_, ii: (ii, 0)),
            pl.BlockSpec((nc, e), lambda nn_, ii: (nn_, 0)),
            pl.BlockSpec((1, nc), lambda nn_, ii: (0, nn_)),
        ],
        out_specs=pl.BlockSpec((bt, tc, e), lambda nn_, ii: (ii, nn_, 0)),
        out_shape=jax.ShapeDtypeStruct((b, ctx, e), jnp.float32),
    )(encoded, dec_w, dec_b2d)


def kernel(context, emb, enc_w, enc_b, dec_w, dec_b):
    b, ctx = context.shape
    _, e = emb.shape
    n_rows = b * ctx
    idx2d = context.reshape(n_rows // IDX_LANES, IDX_LANES)
    gathered = _sc_gather(emb, idx2d, n_rows, e)
    g3 = gathered.reshape(b, ctx, e)
    encoded = _encoder(g3, enc_w.astype(jnp.bfloat16), enc_b.reshape(1, e))
    return _decoder(encoded.astype(jnp.bfloat16), dec_w.astype(jnp.bfloat16),
                    dec_b.reshape(1, ctx * e))


# speedup vs baseline: 1.2219x; 1.2219x over previous
import functools

import jax
import jax.numpy as jnp
from jax import lax
from jax.experimental import pallas as pl
from jax.experimental.pallas import tpu as pltpu
from jax.experimental.pallas import tpu_sc as plsc

NUM_CORES = 2
NUM_SUBCORES = 16
NW = NUM_CORES * NUM_SUBCORES
IDX_LANES = 128
GROUP = 4
ROWS_PER_GROUP = IDX_LANES * GROUP


def _sc_gather(table, idx2d, n_rows, d):
    per_w = n_rows // NW
    idx_rows = per_w // IDX_LANES
    groups = per_w // ROWS_PER_GROUP
    mesh = plsc.VectorSubcoreMesh(core_axis_name="c", subcore_axis_name="s")

    @functools.partial(
        pl.kernel,
        mesh=mesh,
        out_type=jax.ShapeDtypeStruct((n_rows, d), table.dtype),
        scratch_types=[
            pltpu.VMEM((idx_rows, IDX_LANES), jnp.int32),
            pltpu.VMEM((ROWS_PER_GROUP, d), table.dtype),
            pltpu.SemaphoreType.DMA,
        ],
    )
    def gather_kernel(table_hbm, idx_hbm, out_hbm, idx_v, rows_v, sem):
        wid = lax.axis_index("s") * NUM_CORES + lax.axis_index("c")
        row0 = wid * per_w
        pltpu.sync_copy(idx_hbm.at[pl.ds(wid * idx_rows, idx_rows)], idx_v)

        def body(g, carry):
            copies = [
                pltpu.make_async_copy(
                    table_hbm.at[idx_v.at[g * GROUP + b]],
                    rows_v.at[pl.ds(b * IDX_LANES, IDX_LANES)],
                    sem,
                )
                for b in range(GROUP)
            ]
            for c in copies:
                c.start()
            for c in copies:
                c.wait()
            pltpu.sync_copy(
                rows_v, out_hbm.at[pl.ds(row0 + g * ROWS_PER_GROUP, ROWS_PER_GROUP)])
            return carry

        lax.fori_loop(0, groups, body, 0)

    return gather_kernel(table, idx2d)


def _encoder(g3, enc_w, enc_b2d, bt=512, tc=40):
    b, ctx, e = g3.shape
    nb, nk = b // bt, ctx // tc

    def body(g_ref, w_ref, b_ref, out_ref, acc_ref):
        kk = pl.program_id(0)
        ii = pl.program_id(1)
        part = lax.dot_general(
            g_ref[:, 0, :], w_ref[:, 0:e],
            (((1,), (1,)), ((), ())),
            precision=lax.Precision.DEFAULT,
            preferred_element_type=jnp.float32)
        for j in range(1, tc):
            part += lax.dot_general(
                g_ref[:, j, :], w_ref[:, j * e:(j + 1) * e],
                (((1,), (1,)), ((), ())),
                precision=lax.Precision.DEFAULT,
                preferred_element_type=jnp.float32)
        sl = pl.ds(ii * bt, bt)

        @pl.when(kk == 0)
        def _():
            acc_ref[sl, :] = part

        @pl.when(kk > 0)
        def _():
            acc_ref[sl, :] = acc_ref[sl, :] + part

        @pl.when(kk == nk - 1)
        def _():
            out_ref[...] = acc_ref[sl, :] + b_ref[...]

    return pl.pallas_call(
        body,
        grid=(nk, nb),
        in_specs=[
            pl.BlockSpec((bt, tc, e), lambda kk, ii: (ii, kk, 0)),
            pl.BlockSpec((e, tc * e), lambda kk, ii: (0, kk)),
            pl.BlockSpec((1, e), lambda kk, ii: (0, 0)),
        ],
        out_specs=pl.BlockSpec((bt, e), lambda kk, ii: (ii, 0)),
        out_shape=jax.ShapeDtypeStruct((b, e), jnp.float32),
        scratch_shapes=[pltpu.VMEM((b, e), jnp.float32)],
    )(g3, enc_w, enc_b2d)


def _decoder(encoded, dec_w, dec_b2d, bt=512, tc=40):
    b, e = encoded.shape
    k = dec_w.shape[0]
    ctx = k // e
    nb, nn = b // bt, ctx // tc
    nc = tc * e

    def body(enc_ref, w_ref, b_ref, out_ref):
        enc = enc_ref[...]
        for j in range(tc):
            res = lax.dot_general(
                enc, w_ref[j * e:(j + 1) * e, :], (((1,), (1,)), ((), ())),
                preferred_element_type=jnp.float32)
            out_ref[:, j, :] = res + b_ref[0:1, j * e:(j + 1) * e]

    return pl.pallas_call(
        body,
        grid=(nn, nb),
        in_specs=[
            pl.BlockSpec((bt, e), lambda nn_, ii: (ii, 0)),
            pl.BlockSpec((nc, e), lambda nn_, ii: (nn_, 0)),
            pl.BlockSpec((1, nc), lambda nn_, ii: (0, nn_)),
        ],
        out_specs=pl.BlockSpec((bt, tc, e), lambda nn_, ii: (ii, nn_, 0)),
        out_shape=jax.ShapeDtypeStruct((b, ctx, e), jnp.float32),
    )(encoded, dec_w, dec_b2d)


def kernel(context, emb, enc_w, enc_b, dec_w, dec_b):
    b, ctx = context.shape
    _, e = emb.shape
    n_rows = b * ctx
    idx2d = context.reshape(n_rows // IDX_LANES, IDX_LANES)
    gathered = _sc_gather(emb, idx2d, n_rows, e)
    g3 = gathered.reshape(b, ctx, e)
    encoded = _encoder(g3, enc_w, enc_b.reshape(1, e))
    return _decoder(encoded.astype(jnp.bfloat16), dec_w.astype(jnp.bfloat16),
                    dec_b.reshape(1, ctx * e))


# speedup vs baseline: 1.2280x; 1.0050x over previous
import functools

import jax
import jax.numpy as jnp
from jax import lax
from jax.experimental import pallas as pl
from jax.experimental.pallas import tpu as pltpu
from jax.experimental.pallas import tpu_sc as plsc

NUM_CORES = 2
NUM_SUBCORES = 16
NW = NUM_CORES * NUM_SUBCORES
IDX_LANES = 128
GROUP = 4
ROWS_PER_GROUP = IDX_LANES * GROUP


def _sc_gather(table, idx2d, n_rows, d):
    per_w = n_rows // NW
    idx_rows = per_w // IDX_LANES
    groups = per_w // ROWS_PER_GROUP
    mesh = plsc.VectorSubcoreMesh(core_axis_name="c", subcore_axis_name="s")

    @functools.partial(
        pl.kernel,
        mesh=mesh,
        out_type=jax.ShapeDtypeStruct((n_rows, d), table.dtype),
        scratch_types=[
            pltpu.VMEM((idx_rows, IDX_LANES), jnp.int32),
            pltpu.VMEM((ROWS_PER_GROUP, d), table.dtype),
            pltpu.SemaphoreType.DMA,
        ],
    )
    def gather_kernel(table_hbm, idx_hbm, out_hbm, idx_v, rows_v, sem):
        wid = lax.axis_index("s") * NUM_CORES + lax.axis_index("c")
        row0 = wid * per_w
        pltpu.sync_copy(idx_hbm.at[pl.ds(wid * idx_rows, idx_rows)], idx_v)

        def body(g, carry):
            copies = [
                pltpu.make_async_copy(
                    table_hbm.at[idx_v.at[g * GROUP + b]],
                    rows_v.at[pl.ds(b * IDX_LANES, IDX_LANES)],
                    sem,
                )
                for b in range(GROUP)
            ]
            for c in copies:
                c.start()
            for c in copies:
                c.wait()
            pltpu.sync_copy(
                rows_v, out_hbm.at[pl.ds(row0 + g * ROWS_PER_GROUP, ROWS_PER_GROUP)])
            return carry

        lax.fori_loop(0, groups, body, 0)

    return gather_kernel(table, idx2d)


def _encoder(g3, enc_w, enc_b2d, bt=128, tc=200):
    b, ctx, e = g3.shape
    nb, nk = b // bt, ctx // tc

    def body(g_ref, w_ref, b_ref, out_ref, acc_ref):
        kk = pl.program_id(0)
        ii = pl.program_id(1)
        part = lax.dot_general(
            g_ref[:, 0, :], w_ref[:, 0:e],
            (((1,), (1,)), ((), ())),
            precision=lax.Precision.DEFAULT,
            preferred_element_type=jnp.float32)
        for j in range(1, tc):
            part += lax.dot_general(
                g_ref[:, j, :], w_ref[:, j * e:(j + 1) * e],
                (((1,), (1,)), ((), ())),
                precision=lax.Precision.DEFAULT,
                preferred_element_type=jnp.float32)
        sl = pl.ds(ii * bt, bt)

        @pl.when(kk == 0)
        def _():
            acc_ref[sl, :] = part

        @pl.when(kk > 0)
        def _():
            acc_ref[sl, :] = acc_ref[sl, :] + part

        @pl.when(kk == nk - 1)
        def _():
            out_ref[...] = acc_ref[sl, :] + b_ref[...]

    return pl.pallas_call(
        body,
        grid=(nk, nb),
        in_specs=[
            pl.BlockSpec((bt, tc, e), lambda kk, ii: (ii, kk, 0)),
            pl.BlockSpec((e, tc * e), lambda kk, ii: (0, kk)),
            pl.BlockSpec((1, e), lambda kk, ii: (0, 0)),
        ],
        out_specs=pl.BlockSpec((bt, e), lambda kk, ii: (ii, 0)),
        out_shape=jax.ShapeDtypeStruct((b, e), jnp.float32),
        scratch_shapes=[pltpu.VMEM((b, e), jnp.float32)],
    )(g3, enc_w, enc_b2d)


def _decoder(encoded, dec_w, dec_b2d, bt=128, tc=200):
    b, e = encoded.shape
    k = dec_w.shape[0]
    ctx = k // e
    nb, nn = b // bt, ctx // tc
    nc = tc * e

    def body(enc_ref, w_ref, b_ref, out_ref):
        enc = enc_ref[...]
        for j in range(tc):
            res = lax.dot_general(
                enc, w_ref[j * e:(j + 1) * e, :], (((1,), (1,)), ((), ())),
                preferred_element_type=jnp.float32)
            out_ref[:, j, :] = res + b_ref[0:1, j * e:(j + 1) * e]

    return pl.pallas_call(
        body,
        grid=(nn, nb),
        in_specs=[
            pl.BlockSpec((bt, e), lambda nn_, ii: (ii, 0)),
            pl.BlockSpec((nc, e), lambda nn_, ii: (nn_, 0)),
            pl.BlockSpec((1, nc), lambda nn_, ii: (0, nn_)),
        ],
        out_specs=pl.BlockSpec((bt, tc, e), lambda nn_, ii: (ii, nn_, 0)),
        out_shape=jax.ShapeDtypeStruct((b, ctx, e), jnp.float32),
    )(encoded, dec_w, dec_b2d)


def kernel(context, emb, enc_w, enc_b, dec_w, dec_b):
    b, ctx = context.shape
    _, e = emb.shape
    n_rows = b * ctx
    idx2d = context.reshape(n_rows // IDX_LANES, IDX_LANES)
    gathered = _sc_gather(emb, idx2d, n_rows, e)
    g3 = gathered.reshape(b, ctx, e)
    encoded = _encoder(g3, enc_w, enc_b.reshape(1, e))
    return _decoder(encoded.astype(jnp.bfloat16), dec_w.astype(jnp.bfloat16),
                    dec_b.reshape(1, ctx * e))


# speedup vs baseline: 1.2980x; 1.0570x over previous
import functools

import jax
import jax.numpy as jnp
from jax import lax
from jax.experimental import pallas as pl
from jax.experimental.pallas import tpu as pltpu
from jax.experimental.pallas import tpu_sc as plsc

NUM_CORES = 2
NUM_SUBCORES = 16
NW = NUM_CORES * NUM_SUBCORES
IDX_LANES = 128


def _sc_gather(table, idx3d, n_rows, d):
    per_w = n_rows // NW
    idx_rows = per_w // IDX_LANES
    group = next(g for g in (5, 4, 3, 2, 1) if idx_rows % g == 0)
    rows_per_group = IDX_LANES * group
    groups = per_w // rows_per_group
    mesh = plsc.VectorSubcoreMesh(core_axis_name="c", subcore_axis_name="s")

    @functools.partial(
        pl.kernel,
        mesh=mesh,
        out_type=jax.ShapeDtypeStruct((n_rows, d), table.dtype),
        scratch_types=[
            pltpu.VMEM((idx_rows, IDX_LANES), jnp.int32),
            pltpu.VMEM((rows_per_group, d), table.dtype),
            pltpu.SemaphoreType.DMA,
        ],
    )
    def gather_kernel(table_hbm, idx_hbm, out_hbm, idx_v, rows_v, sem):
        wid = lax.axis_index("s") * NUM_CORES + lax.axis_index("c")
        row0 = wid * per_w
        pltpu.sync_copy(idx_hbm.at[wid], idx_v)

        def body(g, carry):
            copies = [
                pltpu.make_async_copy(
                    table_hbm.at[idx_v.at[g * group + b]],
                    rows_v.at[pl.ds(b * IDX_LANES, IDX_LANES)],
                    sem,
                )
                for b in range(group)
            ]
            for c in copies:
                c.start()
            for c in copies:
                c.wait()
            pltpu.sync_copy(
                rows_v, out_hbm.at[pl.ds(row0 + g * rows_per_group, rows_per_group)])
            return carry

        lax.fori_loop(0, groups, body, 0)

    return gather_kernel(table, idx3d)


def _encoder(g3, enc_w, enc_b2d, bt=128, tc=200):
    b, ctx, e = g3.shape
    nb, nk = b // bt, ctx // tc

    def body(g_ref, w_ref, b_ref, out_ref, acc_ref):
        kk = pl.program_id(0)
        ii = pl.program_id(1)
        part = lax.dot_general(
            g_ref[:, 0, :], w_ref[:, 0:e],
            (((1,), (1,)), ((), ())),
            precision=lax.Precision.DEFAULT,
            preferred_element_type=jnp.float32)
        for j in range(1, tc):
            part += lax.dot_general(
                g_ref[:, j, :], w_ref[:, j * e:(j + 1) * e],
                (((1,), (1,)), ((), ())),
                precision=lax.Precision.DEFAULT,
                preferred_element_type=jnp.float32)
        sl = pl.ds(ii * bt, bt)

        @pl.when(kk == 0)
        def _():
            acc_ref[sl, :] = part

        @pl.when(kk > 0)
        def _():
            acc_ref[sl, :] = acc_ref[sl, :] + part

        @pl.when(kk == nk - 1)
        def _():
            out_ref[...] = acc_ref[sl, :] + b_ref[...]

    return pl.pallas_call(
        body,
        grid=(nk, nb),
        in_specs=[
            pl.BlockSpec((bt, tc, e), lambda kk, ii: (ii, kk, 0)),
            pl.BlockSpec((e, tc * e), lambda kk, ii: (0, kk)),
            pl.BlockSpec((1, e), lambda kk, ii: (0, 0)),
        ],
        out_specs=pl.BlockSpec((bt, e), lambda kk, ii: (ii, 0)),
        out_shape=jax.ShapeDtypeStruct((b, e), jnp.float32),
        scratch_shapes=[pltpu.VMEM((b, e), jnp.float32)],
    )(g3, enc_w, enc_b2d)


def _decoder_chunk(encoded, dec_w, dec_b2d, b_total, blk0, prev, bt=128):
    bch, e = encoded.shape
    k = dec_w.shape[0]
    ctx = k // e
    nb = bch // bt

    def body(enc_ref, w_ref, b_ref, *refs):
        out_ref = refs[-1]
        enc = enc_ref[...]
        for j in range(ctx):
            res = lax.dot_general(
                enc, w_ref[j * e:(j + 1) * e, :], (((1,), (1,)), ((), ())),
                preferred_element_type=jnp.float32)
            out_ref[:, j, :] = res + b_ref[0:1, j * e:(j + 1) * e]

    in_specs = [
        pl.BlockSpec((bt, e), lambda ii: (ii, 0)),
        pl.BlockSpec((k, e), lambda ii: (0, 0)),
        pl.BlockSpec((1, k), lambda ii: (0, 0)),
    ]
    args = [encoded, dec_w, dec_b2d]
    alias = {}
    if prev is not None:
        in_specs.append(pl.BlockSpec(memory_space=pl.ANY))
        args.append(prev)
        alias = {3: 0}

    return pl.pallas_call(
        body,
        grid=(nb,),
        in_specs=in_specs,
        out_specs=pl.BlockSpec((bt, ctx, e), lambda ii: (blk0 + ii, 0, 0)),
        out_shape=jax.ShapeDtypeStruct((b_total, ctx, e), jnp.float32),
        input_output_aliases=alias,
    )(*args)


def kernel(context, emb, enc_w, enc_b, dec_w, dec_b):
    b, ctx = context.shape
    _, e = emb.shape
    nch = 4
    bch = b // nch
    bt = 128
    enc_b2d = enc_b.reshape(1, e)
    dec_b2d = dec_b.reshape(1, ctx * e)
    dec_w_bf = dec_w.astype(jnp.bfloat16)

    out = None
    for c in range(nch):
        idx_c = context[c * bch:(c + 1) * bch]
        idx3d = idx_c.reshape(NW, bch * ctx // NW // IDX_LANES, IDX_LANES)
        gathered = _sc_gather(emb, idx3d, bch * ctx, e)
        g3 = gathered.reshape(bch, ctx, e)
        encoded = _encoder(g3, enc_w, enc_b2d, bt=bt)
        out = _decoder_chunk(encoded.astype(jnp.bfloat16), dec_w_bf, dec_b2d,
                             b, c * (bch // bt), out, bt=bt)
    return out
